# Initial kernel scaffold; baseline (speedup 1.0000x reference)
import jax
import jax.numpy as jnp
from jax.experimental import pallas as pl


def kernel(x, emb, W_ih0, W_hh0, b_ih0, b_hh0, W_ih1, W_hh1, b_ih1, b_hh1, linW, linb):
    raise NotImplementedError("write your pallas kernel here")


# speedup vs baseline: 1.4894x; 1.4894x over previous
import functools

import jax
import jax.numpy as jnp
from jax import lax
from jax.experimental import pallas as pl
from jax.experimental.pallas import tpu as pltpu
from jax.experimental.pallas import tpu_sc as plsc


def _make_gather(V, D, N):
    info = plsc.get_sparse_core_info()
    NC, NS = info.num_cores, info.num_subcores
    NW = NC * NS
    per_w = N // NW
    CH = min(128, per_w)
    n_ch = per_w // CH
    mesh = plsc.VectorSubcoreMesh(core_axis_name="c", subcore_axis_name="s")

    @functools.partial(
        pl.kernel,
        mesh=mesh,
        compiler_params=pltpu.CompilerParams(use_tc_tiling_on_sc=False),
        out_type=jax.ShapeDtypeStruct((N, D), jnp.float32),
        scratch_types=[
            pltpu.VMEM((n_ch, CH), jnp.int32),
            pltpu.VMEM((CH, D), jnp.float32),
            pltpu.SemaphoreType.DMA,
        ],
    )
    def gather(table_hbm, idx_hbm, out_hbm, idx_v, rows_v, sem):
        wid = lax.axis_index("s") * NC + lax.axis_index("c")
        base = wid * per_w
        for ch in range(n_ch):
            pltpu.sync_copy(idx_hbm.at[pl.ds(base + ch * CH, CH)], idx_v.at[ch])
        for ch in range(n_ch):
            pltpu.async_copy(
                table_hbm.at[idx_v.at[ch]], rows_v, sem
            ).wait()
            pltpu.sync_copy(rows_v, out_hbm.at[pl.ds(base + ch * CH, CH)])

    return gather


def _proj_body(e_ref, w_ref, b_ref, o_ref):
    o_ref[...] = (
        jnp.dot(e_ref[...], w_ref[...], preferred_element_type=jnp.float32)
        + b_ref[...]
    )


def _projection(e, wT, bias, row_block=512):
    N, D = e.shape
    G4 = wT.shape[1]
    grid = (N // row_block,)
    return pl.pallas_call(
        _proj_body,
        grid=grid,
        in_specs=[
            pl.BlockSpec((row_block, D), lambda i: (i, 0)),
            pl.BlockSpec((D, G4), lambda i: (0, 0)),
            pl.BlockSpec((1, G4), lambda i: (0, 0)),
        ],
        out_specs=pl.BlockSpec((row_block, G4), lambda i: (i, 0)),
        out_shape=jax.ShapeDtypeStruct((N, G4), jnp.float32),
    )(e, wT, bias)


def _rec_body(xp_ref, whh_ref, lw_ref, lb_ref, out_ref, h_ref, c_ref, *, H):
    t = pl.program_id(0)

    @pl.when(t == 0)
    def _init():
        h_ref[...] = jnp.zeros_like(h_ref)
        c_ref[...] = jnp.zeros_like(c_ref)

    g = xp_ref[0] + jnp.dot(
        h_ref[...], whh_ref[...], preferred_element_type=jnp.float32
    )
    i = jax.nn.sigmoid(g[:, :H])
    f = jax.nn.sigmoid(g[:, H : 2 * H])
    gg = jnp.tanh(g[:, 2 * H : 3 * H])
    o = jax.nn.sigmoid(g[:, 3 * H :])
    c = f * c_ref[...] + i * gg
    h = o * jnp.tanh(c)
    c_ref[...] = c
    h_ref[...] = h

    @pl.when(t == pl.num_programs(0) - 1)
    def _head():
        out_ref[...] = (
            jnp.dot(h, lw_ref[...], preferred_element_type=jnp.float32)
            + lb_ref[...]
        )


def _recurrence(xp, whhT, lwT, lb):
    S, B, G4 = xp.shape
    H = G4 // 4
    C = lwT.shape[1]
    return pl.pallas_call(
        functools.partial(_rec_body, H=H),
        grid=(S,),
        in_specs=[
            pl.BlockSpec((1, B, G4), lambda t: (t, 0, 0)),
            pl.BlockSpec((H, G4), lambda t: (0, 0)),
            pl.BlockSpec((H, C), lambda t: (0, 0)),
            pl.BlockSpec((1, C), lambda t: (0, 0)),
        ],
        out_specs=pl.BlockSpec((B, C), lambda t: (0, 0)),
        out_shape=jax.ShapeDtypeStruct((B, C), jnp.float32),
        scratch_shapes=[
            pltpu.VMEM((B, H), jnp.float32),
            pltpu.VMEM((B, H), jnp.float32),
        ],
    )(xp, whhT, lwT, lb)


def kernel(x, emb, W_ih0, W_hh0, b_ih0, b_hh0,
           W_ih1, W_hh1, b_ih1, b_hh1, linW, linb):
    B, S = x.shape
    V, D = emb.shape
    G4, H = W_hh0.shape[0], W_hh0.shape[1]
    C = linW.shape[0]

    idx = jnp.transpose(x).reshape(-1)
    Dp = 304
    embp = jnp.pad(emb, ((0, 0), (0, Dp - D)))
    e = _make_gather(V, Dp, S * B)(embp, idx)

    wT = jnp.pad(jnp.transpose(W_ih0), ((0, Dp - D), (0, 0)))
    bias = (b_ih0 + b_hh0).reshape(1, G4)
    xp = _projection(e, wT, bias).reshape(S, B, G4)

    whhT = jnp.transpose(W_hh0)
    lwT = jnp.transpose(linW)
    out = _recurrence(xp, whhT, lwT, linb.reshape(1, C))
    return out


# speedup vs baseline: 1.5866x; 1.0652x over previous
import functools

import jax
import jax.numpy as jnp
from jax import lax
from jax.experimental import pallas as pl
from jax.experimental.pallas import tpu as pltpu
from jax.experimental.pallas import tpu_sc as plsc


def _make_gather(V, D, N):
    info = plsc.get_sparse_core_info()
    NC, NS = info.num_cores, info.num_subcores
    NW = NC * NS
    per_w = N // NW
    CH = min(128, per_w)
    n_ch = per_w // CH
    mesh = plsc.VectorSubcoreMesh(core_axis_name="c", subcore_axis_name="s")

    @functools.partial(
        pl.kernel,
        mesh=mesh,
        compiler_params=pltpu.CompilerParams(use_tc_tiling_on_sc=False),
        out_type=jax.ShapeDtypeStruct((N, D), jnp.float32),
        scratch_types=[
            pltpu.VMEM((n_ch, CH), jnp.int32),
            pltpu.VMEM((CH, D), jnp.float32),
            pltpu.SemaphoreType.DMA,
        ],
    )
    def gather(table_hbm, idx_hbm, out_hbm, idx_v, rows_v, sem):
        wid = lax.axis_index("s") * NC + lax.axis_index("c")
        base = wid * per_w
        for ch in range(n_ch):
            pltpu.sync_copy(idx_hbm.at[pl.ds(base + ch * CH, CH)], idx_v.at[ch])
        for ch in range(n_ch):
            pltpu.async_copy(
                table_hbm.at[idx_v.at[ch]], rows_v, sem
            ).wait()
            pltpu.sync_copy(rows_v, out_hbm.at[pl.ds(base + ch * CH, CH)])

    return gather


def _proj_body(p_ref, q_ref, w_ref, b_ref, o_ref, *, D):
    e = jnp.where(q_ref[...] > 0, p_ref[:, D:], p_ref[:, :D])
    o_ref[...] = (
        jnp.dot(e, w_ref[...], preferred_element_type=jnp.float32)
        + b_ref[...]
    )


def _projection(pairs, q, wT, bias, row_block=512):
    N, D2 = pairs.shape
    D = D2 // 2
    G4 = wT.shape[1]
    grid = (N // row_block,)
    return pl.pallas_call(
        functools.partial(_proj_body, D=D),
        grid=grid,
        in_specs=[
            pl.BlockSpec((row_block, D2), lambda i: (i, 0)),
            pl.BlockSpec((row_block, 1), lambda i: (i, 0)),
            pl.BlockSpec((D, G4), lambda i: (0, 0)),
            pl.BlockSpec((1, G4), lambda i: (0, 0)),
        ],
        out_specs=pl.BlockSpec((row_block, G4), lambda i: (i, 0)),
        out_shape=jax.ShapeDtypeStruct((N, G4), jnp.float32),
    )(pairs, q, wT, bias)


def _rec_body(xp_ref, whh_ref, lw_ref, lb_ref, out_ref, h_ref, c_ref, *, H):
    t = pl.program_id(0)

    @pl.when(t == 0)
    def _init():
        h_ref[...] = jnp.zeros_like(h_ref)
        c_ref[...] = jnp.zeros_like(c_ref)

    g = xp_ref[0] + jnp.dot(
        h_ref[...], whh_ref[...], preferred_element_type=jnp.float32
    )
    i = jax.nn.sigmoid(g[:, :H])
    f = jax.nn.sigmoid(g[:, H : 2 * H])
    gg = jnp.tanh(g[:, 2 * H : 3 * H])
    o = jax.nn.sigmoid(g[:, 3 * H :])
    c = f * c_ref[...] + i * gg
    h = o * jnp.tanh(c)
    c_ref[...] = c
    h_ref[...] = h

    @pl.when(t == pl.num_programs(0) - 1)
    def _head():
        out_ref[...] = (
            jnp.dot(h, lw_ref[...], preferred_element_type=jnp.float32)
            + lb_ref[...]
        )


def _recurrence(xp, whhT, lwT, lb):
    S, B, G4 = xp.shape
    H = G4 // 4
    C = lwT.shape[1]
    return pl.pallas_call(
        functools.partial(_rec_body, H=H),
        grid=(S,),
        in_specs=[
            pl.BlockSpec((1, B, G4), lambda t: (t, 0, 0)),
            pl.BlockSpec((H, G4), lambda t: (0, 0)),
            pl.BlockSpec((H, C), lambda t: (0, 0)),
            pl.BlockSpec((1, C), lambda t: (0, 0)),
        ],
        out_specs=pl.BlockSpec((B, C), lambda t: (0, 0)),
        out_shape=jax.ShapeDtypeStruct((B, C), jnp.float32),
        scratch_shapes=[
            pltpu.VMEM((B, H), jnp.float32),
            pltpu.VMEM((B, H), jnp.float32),
        ],
    )(xp, whhT, lwT, lb)


def kernel(x, emb, W_ih0, W_hh0, b_ih0, b_hh0,
           W_ih1, W_hh1, b_ih1, b_hh1, linW, linb):
    B, S = x.shape
    V, D = emb.shape
    G4, H = W_hh0.shape[0], W_hh0.shape[1]
    C = linW.shape[0]

    idx = jnp.transpose(x).reshape(-1)
    pairs_view = emb.reshape(V // 2, 2 * D)
    pairs = _make_gather(V // 2, 2 * D, S * B)(pairs_view, idx // 2)
    q = (idx % 2).astype(jnp.float32).reshape(-1, 1)

    wT = jnp.transpose(W_ih0)
    bias = (b_ih0 + b_hh0).reshape(1, G4)
    xp = _projection(pairs, q, wT, bias).reshape(S, B, G4)

    whhT = jnp.transpose(W_hh0)
    lwT = jnp.transpose(linW)
    out = _recurrence(xp, whhT, lwT, linb.reshape(1, C))
    return out


# speedup vs baseline: 3.1122x; 1.9616x over previous
import functools

import jax
import jax.numpy as jnp
from jax import lax
from jax.experimental import pallas as pl
from jax.experimental.pallas import tpu as pltpu
from jax.experimental.pallas import tpu_sc as plsc


def _make_gather3(V, N):
    info = plsc.get_sparse_core_info()
    NC, NS = info.num_cores, info.num_subcores
    NW = NC * NS
    per_w = N // NW
    CH = min(128, per_w)
    n_ch = per_w // CH
    mesh = plsc.VectorSubcoreMesh(core_axis_name="c", subcore_axis_name="s")

    @functools.partial(
        pl.kernel,
        mesh=mesh,
        compiler_params=pltpu.CompilerParams(use_tc_tiling_on_sc=False),
        out_type=[jax.ShapeDtypeStruct((N, 128), jnp.float32)] * 3,
        scratch_types=[
            pltpu.VMEM((n_ch, CH), jnp.int32),
            pltpu.VMEM((CH, 128), jnp.float32),
            pltpu.VMEM((CH, 128), jnp.float32),
            pltpu.VMEM((CH, 128), jnp.float32),
            pltpu.SemaphoreType.DMA,
            pltpu.SemaphoreType.DMA,
            pltpu.SemaphoreType.DMA,
        ],
    )
    def gather(l0, l1, l2, idx_hbm, o0, o1, o2, idx_v, b0, b1, b2, s0, s1, s2):
        wid = lax.axis_index("s") * NC + lax.axis_index("c")
        base = wid * per_w
        for ch in range(n_ch):
            pltpu.sync_copy(idx_hbm.at[pl.ds(base + ch * CH, CH)], idx_v.at[ch])
        for ch in range(n_ch):
            c0 = pltpu.async_copy(l0.at[idx_v.at[ch]], b0, s0)
            c1 = pltpu.async_copy(l1.at[idx_v.at[ch]], b1, s1)
            c2 = pltpu.async_copy(l2.at[idx_v.at[ch]], b2, s2)
            c0.wait()
            c1.wait()
            c2.wait()
            pltpu.sync_copy(b0, o0.at[pl.ds(base + ch * CH, CH)])
            pltpu.sync_copy(b1, o1.at[pl.ds(base + ch * CH, CH)])
            pltpu.sync_copy(b2, o2.at[pl.ds(base + ch * CH, CH)])

    return gather


def _proj_body(e0_ref, e1_ref, e2_ref, w0_ref, w1_ref, w2_ref, b_ref, o_ref):
    acc = jnp.dot(e0_ref[...], w0_ref[...], preferred_element_type=jnp.float32)
    acc += jnp.dot(e1_ref[...], w1_ref[...], preferred_element_type=jnp.float32)
    acc += jnp.dot(e2_ref[...], w2_ref[...], preferred_element_type=jnp.float32)
    o_ref[...] = acc + b_ref[...]


def _projection(planes, ws, bias, row_block=512):
    N = planes[0].shape[0]
    G4 = ws[0].shape[1]
    grid = (N // row_block,)
    return pl.pallas_call(
        _proj_body,
        grid=grid,
        in_specs=[pl.BlockSpec((row_block, 128), lambda i: (i, 0))] * 3
        + [pl.BlockSpec((128, G4), lambda i: (0, 0))] * 3
        + [pl.BlockSpec((1, G4), lambda i: (0, 0))],
        out_specs=pl.BlockSpec((row_block, G4), lambda i: (i, 0)),
        out_shape=jax.ShapeDtypeStruct((N, G4), jnp.float32),
    )(*planes, *ws, bias)


def _rec_body(xp_ref, whh_ref, lw_ref, lb_ref, out_ref, h_ref, c_ref, *, H):
    t = pl.program_id(0)

    @pl.when(t == 0)
    def _init():
        h_ref[...] = jnp.zeros_like(h_ref)
        c_ref[...] = jnp.zeros_like(c_ref)

    g = xp_ref[0] + jnp.dot(
        h_ref[...], whh_ref[...], preferred_element_type=jnp.float32
    )
    i = jax.nn.sigmoid(g[:, :H])
    f = jax.nn.sigmoid(g[:, H : 2 * H])
    gg = jnp.tanh(g[:, 2 * H : 3 * H])
    o = jax.nn.sigmoid(g[:, 3 * H :])
    c = f * c_ref[...] + i * gg
    h = o * jnp.tanh(c)
    c_ref[...] = c
    h_ref[...] = h

    @pl.when(t == pl.num_programs(0) - 1)
    def _head():
        out_ref[...] = (
            jnp.dot(h, lw_ref[...], preferred_element_type=jnp.float32)
            + lb_ref[...]
        )


def _recurrence(xp, whhT, lwT, lb):
    S, B, G4 = xp.shape
    H = G4 // 4
    C = lwT.shape[1]
    return pl.pallas_call(
        functools.partial(_rec_body, H=H),
        grid=(S,),
        in_specs=[
            pl.BlockSpec((1, B, G4), lambda t: (t, 0, 0)),
            pl.BlockSpec((H, G4), lambda t: (0, 0)),
            pl.BlockSpec((H, C), lambda t: (0, 0)),
            pl.BlockSpec((1, C), lambda t: (0, 0)),
        ],
        out_specs=pl.BlockSpec((B, C), lambda t: (0, 0)),
        out_shape=jax.ShapeDtypeStruct((B, C), jnp.float32),
        scratch_shapes=[
            pltpu.VMEM((B, H), jnp.float32),
            pltpu.VMEM((B, H), jnp.float32),
        ],
    )(xp, whhT, lwT, lb)


def kernel(x, emb, W_ih0, W_hh0, b_ih0, b_hh0,
           W_ih1, W_hh1, b_ih1, b_hh1, linW, linb):
    B, S = x.shape
    V, D = emb.shape
    G4, H = W_hh0.shape[0], W_hh0.shape[1]
    C = linW.shape[0]

    idx = jnp.transpose(x).reshape(-1)
    L0 = lax.slice(emb, (0, 0), (V, 128))
    L1 = lax.slice(emb, (0, 128), (V, 256))
    L2 = lax.slice(emb, (0, 172), (V, 300))
    O0, O1, O2 = _make_gather3(V, S * B)(L0, L1, L2, idx)

    wT = jnp.transpose(W_ih0)
    W0 = wT[0:128]
    W1 = wT[128:256]
    W2 = jnp.concatenate([jnp.zeros((84, G4), wT.dtype), wT[256:300]], axis=0)
    bias = (b_ih0 + b_hh0).reshape(1, G4)
    xp = _projection((O0, O1, O2), (W0, W1, W2), bias).reshape(S, B, G4)

    whhT = jnp.transpose(W_hh0)
    lwT = jnp.transpose(linW)
    out = _recurrence(xp, whhT, lwT, linb.reshape(1, C))
    return out


# speedup vs baseline: 4.2706x; 1.3722x over previous
import functools

import jax
import jax.numpy as jnp
from jax import lax
from jax.experimental import pallas as pl
from jax.experimental.pallas import tpu as pltpu
from jax.experimental.pallas import tpu_sc as plsc


def _make_gather3(V, N):
    info = plsc.get_sparse_core_info()
    NC, NS = info.num_cores, info.num_subcores
    NW = NC * NS
    per_w = N // NW
    CH = min(128, per_w)
    n_ch = per_w // CH
    mesh = plsc.VectorSubcoreMesh(core_axis_name="c", subcore_axis_name="s")

    @functools.partial(
        pl.kernel,
        mesh=mesh,
        compiler_params=pltpu.CompilerParams(use_tc_tiling_on_sc=False),
        out_type=[jax.ShapeDtypeStruct((N, 128), jnp.float32)] * 3,
        scratch_types=[
            pltpu.VMEM((n_ch, CH), jnp.int32),
            pltpu.VMEM((CH, 128), jnp.float32),
            pltpu.VMEM((CH, 128), jnp.float32),
            pltpu.VMEM((CH, 128), jnp.float32),
            pltpu.SemaphoreType.DMA,
            pltpu.SemaphoreType.DMA,
            pltpu.SemaphoreType.DMA,
        ],
    )
    def gather(l0, l1, l2, idx_hbm, o0, o1, o2, idx_v, b0, b1, b2, s0, s1, s2):
        wid = lax.axis_index("s") * NC + lax.axis_index("c")
        base = wid * per_w
        for ch in range(n_ch):
            pltpu.sync_copy(idx_hbm.at[pl.ds(base + ch * CH, CH)], idx_v.at[ch])
        for ch in range(n_ch):
            c0 = pltpu.async_copy(l0.at[idx_v.at[ch]], b0, s0)
            c1 = pltpu.async_copy(l1.at[idx_v.at[ch]], b1, s1)
            c2 = pltpu.async_copy(l2.at[idx_v.at[ch]], b2, s2)
            c0.wait()
            c1.wait()
            c2.wait()
            pltpu.sync_copy(b0, o0.at[pl.ds(base + ch * CH, CH)])
            pltpu.sync_copy(b1, o1.at[pl.ds(base + ch * CH, CH)])
            pltpu.sync_copy(b2, o2.at[pl.ds(base + ch * CH, CH)])

    return gather


def _detile_body(et_ref, o0_ref, o1_ref, o2_ref):
    t = et_ref[...]
    o0_ref[...] = jnp.transpose(t[0:128, :])
    o1_ref[...] = jnp.transpose(t[128:256, :])
    o2_ref[...] = jnp.transpose(t[172:300, :])


def _detile(embT, row_block=2048):
    D, V = embT.shape
    grid = (pl.cdiv(V, row_block),)
    return pl.pallas_call(
        _detile_body,
        grid=grid,
        in_specs=[pl.BlockSpec((D, row_block), lambda i: (0, i))],
        out_specs=[pl.BlockSpec((row_block, 128), lambda i: (i, 0))] * 3,
        out_shape=[jax.ShapeDtypeStruct((V, 128), jnp.float32)] * 3,
    )(embT)


def _proj_body(e0_ref, e1_ref, e2_ref, w0_ref, w1_ref, w2_ref, b_ref, o_ref):
    acc = jnp.dot(e0_ref[...], w0_ref[...], preferred_element_type=jnp.float32)
    acc += jnp.dot(e1_ref[...], w1_ref[...], preferred_element_type=jnp.float32)
    acc += jnp.dot(e2_ref[...], w2_ref[...], preferred_element_type=jnp.float32)
    o_ref[...] = acc + b_ref[...]


def _projection(planes, ws, bias, row_block=512):
    N = planes[0].shape[0]
    G4 = ws[0].shape[1]
    grid = (N // row_block,)
    return pl.pallas_call(
        _proj_body,
        grid=grid,
        in_specs=[pl.BlockSpec((row_block, 128), lambda i: (i, 0))] * 3
        + [pl.BlockSpec((128, G4), lambda i: (0, 0))] * 3
        + [pl.BlockSpec((1, G4), lambda i: (0, 0))],
        out_specs=pl.BlockSpec((row_block, G4), lambda i: (i, 0)),
        out_shape=jax.ShapeDtypeStruct((N, G4), jnp.float32),
    )(*planes, *ws, bias)


def _rec_body(xp_ref, whh_ref, lw_ref, lb_ref, out_ref, h_ref, c_ref, *, H):
    t = pl.program_id(0)

    @pl.when(t == 0)
    def _init():
        h_ref[...] = jnp.zeros_like(h_ref)
        c_ref[...] = jnp.zeros_like(c_ref)

    g = xp_ref[...] + jnp.dot(
        h_ref[...], whh_ref[...], preferred_element_type=jnp.float32
    )
    i = jax.nn.sigmoid(g[:, :H])
    f = jax.nn.sigmoid(g[:, H : 2 * H])
    gg = jnp.tanh(g[:, 2 * H : 3 * H])
    o = jax.nn.sigmoid(g[:, 3 * H :])
    c = f * c_ref[...] + i * gg
    h = o * jnp.tanh(c)
    c_ref[...] = c
    h_ref[...] = h

    @pl.when(t == pl.num_programs(0) - 1)
    def _head():
        out_ref[...] = (
            jnp.dot(h, lw_ref[...], preferred_element_type=jnp.float32)
            + lb_ref[...]
        )


def _recurrence(xp, whhT, lwT, lb, B):
    N, G4 = xp.shape
    S = N // B
    H = G4 // 4
    C = lwT.shape[1]
    return pl.pallas_call(
        functools.partial(_rec_body, H=H),
        grid=(S,),
        in_specs=[
            pl.BlockSpec((B, G4), lambda t: (t, 0)),
            pl.BlockSpec((H, G4), lambda t: (0, 0)),
            pl.BlockSpec((H, C), lambda t: (0, 0)),
            pl.BlockSpec((1, C), lambda t: (0, 0)),
        ],
        out_specs=pl.BlockSpec((B, C), lambda t: (0, 0)),
        out_shape=jax.ShapeDtypeStruct((B, C), jnp.float32),
        scratch_shapes=[
            pltpu.VMEM((B, H), jnp.float32),
            pltpu.VMEM((B, H), jnp.float32),
        ],
    )(xp, whhT, lwT, lb)


def kernel(x, emb, W_ih0, W_hh0, b_ih0, b_hh0,
           W_ih1, W_hh1, b_ih1, b_hh1, linW, linb):
    B, S = x.shape
    V, D = emb.shape
    G4, H = W_hh0.shape[0], W_hh0.shape[1]
    C = linW.shape[0]

    idx = jnp.transpose(x).reshape(-1)
    L0, L1, L2 = _detile(jnp.transpose(emb))
    O0, O1, O2 = _make_gather3(V, S * B)(L0, L1, L2, idx)

    wT = jnp.transpose(W_ih0)
    W0 = wT[0:128]
    W1 = wT[128:256]
    W2 = jnp.concatenate([jnp.zeros((84, G4), wT.dtype), wT[256:300]], axis=0)
    bias = (b_ih0 + b_hh0).reshape(1, G4)
    xp = _projection((O0, O1, O2), (W0, W1, W2), bias)

    whhT = jnp.transpose(W_hh0)
    lwT = jnp.transpose(linW)
    out = _recurrence(xp, whhT, lwT, linb.reshape(1, C), B)
    return out


# speedup vs baseline: 4.4455x; 1.0409x over previous
import functools

import jax
import jax.numpy as jnp
from jax import lax
from jax.experimental import pallas as pl
from jax.experimental.pallas import tpu as pltpu
from jax.experimental.pallas import tpu_sc as plsc


def _make_gather3(V, N):
    info = plsc.get_sparse_core_info()
    NC, NS = info.num_cores, info.num_subcores
    NW = NC * NS
    per_w = N // NW
    CH = min(128, per_w)
    n_ch = per_w // CH
    mesh = plsc.VectorSubcoreMesh(core_axis_name="c", subcore_axis_name="s")

    @functools.partial(
        pl.kernel,
        mesh=mesh,
        compiler_params=pltpu.CompilerParams(use_tc_tiling_on_sc=False),
        out_type=[jax.ShapeDtypeStruct((N, 128), jnp.float32)] * 3,
        scratch_types=[
            pltpu.VMEM((n_ch, CH), jnp.int32),
            pltpu.VMEM((CH, 128), jnp.float32),
            pltpu.VMEM((CH, 128), jnp.float32),
            pltpu.VMEM((CH, 128), jnp.float32),
            pltpu.SemaphoreType.DMA,
            pltpu.SemaphoreType.DMA,
            pltpu.SemaphoreType.DMA,
        ],
    )
    def gather(l0, l1, l2, idx_hbm, o0, o1, o2, idx_v, b0, b1, b2, s0, s1, s2):
        wid = lax.axis_index("s") * NC + lax.axis_index("c")
        base = wid * per_w
        for ch in range(n_ch):
            pltpu.sync_copy(idx_hbm.at[pl.ds(base + ch * CH, CH)], idx_v.at[ch])
        for ch in range(n_ch):
            c0 = pltpu.async_copy(l0.at[idx_v.at[ch]], b0, s0)
            c1 = pltpu.async_copy(l1.at[idx_v.at[ch]], b1, s1)
            c2 = pltpu.async_copy(l2.at[idx_v.at[ch]], b2, s2)
            c0.wait()
            c1.wait()
            c2.wait()
            pltpu.sync_copy(b0, o0.at[pl.ds(base + ch * CH, CH)])
            pltpu.sync_copy(b1, o1.at[pl.ds(base + ch * CH, CH)])
            pltpu.sync_copy(b2, o2.at[pl.ds(base + ch * CH, CH)])

    return gather


def _detile_body(et_ref, o0_ref, o1_ref, o2_ref):
    t = et_ref[...]
    o0_ref[...] = jnp.transpose(t[0:128, :])
    o1_ref[...] = jnp.transpose(t[128:256, :])
    o2_ref[...] = jnp.transpose(t[172:300, :])


def _detile(embT, row_block=2048):
    D, V = embT.shape
    grid = (pl.cdiv(V, row_block),)
    return pl.pallas_call(
        _detile_body,
        grid=grid,
        in_specs=[pl.BlockSpec((D, row_block), lambda i: (0, i))],
        out_specs=[pl.BlockSpec((row_block, 128), lambda i: (i, 0))] * 3,
        out_shape=[jax.ShapeDtypeStruct((V, 128), jnp.float32)] * 3,
    )(embT)


def _proj_body(e0_ref, e1_ref, e2_ref, w0_ref, w1_ref, w2_ref, b_ref, o_ref):
    bf = jnp.bfloat16
    acc = jnp.dot(e0_ref[...].astype(bf), w0_ref[...],
                  preferred_element_type=jnp.float32)
    acc += jnp.dot(e1_ref[...].astype(bf), w1_ref[...],
                   preferred_element_type=jnp.float32)
    acc += jnp.dot(e2_ref[...].astype(bf), w2_ref[...],
                   preferred_element_type=jnp.float32)
    o_ref[...] = (acc + b_ref[...]).astype(bf)


def _projection(planes, ws, bias, row_block=512):
    N = planes[0].shape[0]
    G4 = ws[0].shape[1]
    grid = (N // row_block,)
    return pl.pallas_call(
        _proj_body,
        grid=grid,
        in_specs=[pl.BlockSpec((row_block, 128), lambda i: (i, 0))] * 3
        + [pl.BlockSpec((128, G4), lambda i: (0, 0))] * 3
        + [pl.BlockSpec((1, G4), lambda i: (0, 0))],
        out_specs=pl.BlockSpec((row_block, G4), lambda i: (i, 0)),
        out_shape=jax.ShapeDtypeStruct((N, G4), jnp.bfloat16),
    )(*planes, *ws, bias)


def _rec_body(xp_ref, whh_ref, lw_ref, lb_ref, out_ref, h_ref, c_ref, *, H):
    t = pl.program_id(0)

    @pl.when(t == 0)
    def _init():
        h_ref[...] = jnp.zeros_like(h_ref)
        c_ref[...] = jnp.zeros_like(c_ref)

    g = xp_ref[...].astype(jnp.float32) + jnp.dot(
        h_ref[...].astype(jnp.bfloat16), whh_ref[...],
        preferred_element_type=jnp.float32,
    )
    i = jax.nn.sigmoid(g[:, :H])
    f = jax.nn.sigmoid(g[:, H : 2 * H])
    gg = jnp.tanh(g[:, 2 * H : 3 * H])
    o = jax.nn.sigmoid(g[:, 3 * H :])
    c = f * c_ref[...] + i * gg
    h = o * jnp.tanh(c)
    c_ref[...] = c
    h_ref[...] = h

    @pl.when(t == pl.num_programs(0) - 1)
    def _head():
        out_ref[...] = (
            jnp.dot(h, lw_ref[...], preferred_element_type=jnp.float32)
            + lb_ref[...]
        )


def _recurrence(xp, whhT, lwT, lb, B):
    N, G4 = xp.shape
    S = N // B
    H = G4 // 4
    C = lwT.shape[1]
    return pl.pallas_call(
        functools.partial(_rec_body, H=H),
        grid=(S,),
        in_specs=[
            pl.BlockSpec((B, G4), lambda t: (t, 0)),
            pl.BlockSpec((H, G4), lambda t: (0, 0)),
            pl.BlockSpec((H, C), lambda t: (0, 0)),
            pl.BlockSpec((1, C), lambda t: (0, 0)),
        ],
        out_specs=pl.BlockSpec((B, C), lambda t: (0, 0)),
        out_shape=jax.ShapeDtypeStruct((B, C), jnp.float32),
        scratch_shapes=[
            pltpu.VMEM((B, H), jnp.float32),
            pltpu.VMEM((B, H), jnp.float32),
        ],
    )(xp, whhT, lwT, lb)


def kernel(x, emb, W_ih0, W_hh0, b_ih0, b_hh0,
           W_ih1, W_hh1, b_ih1, b_hh1, linW, linb):
    B, S = x.shape
    V, D = emb.shape
    G4, H = W_hh0.shape[0], W_hh0.shape[1]
    C = linW.shape[0]

    idx = jnp.transpose(x).reshape(-1)
    L0, L1, L2 = _detile(jnp.transpose(emb))
    O0, O1, O2 = _make_gather3(V, S * B)(L0, L1, L2, idx)

    wT = jnp.transpose(W_ih0)
    W0 = wT[0:128].astype(jnp.bfloat16)
    W1 = wT[128:256].astype(jnp.bfloat16)
    W2 = jnp.concatenate(
        [jnp.zeros((84, G4), wT.dtype), wT[256:300]], axis=0
    ).astype(jnp.bfloat16)
    bias = (b_ih0 + b_hh0).reshape(1, G4)
    xp = _projection((O0, O1, O2), (W0, W1, W2), bias)

    whhT = jnp.transpose(W_hh0).astype(jnp.bfloat16)
    lwT = jnp.transpose(linW)
    out = _recurrence(xp, whhT, lwT, linb.reshape(1, C), B)
    return out


# speedup vs baseline: 5.3198x; 1.1967x over previous
import functools

import jax
import jax.numpy as jnp
from jax import lax
from jax.experimental import pallas as pl
from jax.experimental.pallas import tpu as pltpu
from jax.experimental.pallas import tpu_sc as plsc


def _make_gather3(V, N):
    info = plsc.get_sparse_core_info()
    NC, NS = info.num_cores, info.num_subcores
    NW = NC * NS
    per_w = N // NW
    CH = min(128, per_w)
    n_ch = per_w // CH
    mesh = plsc.VectorSubcoreMesh(core_axis_name="c", subcore_axis_name="s")

    @functools.partial(
        pl.kernel,
        mesh=mesh,
        compiler_params=pltpu.CompilerParams(use_tc_tiling_on_sc=False),
        out_type=[jax.ShapeDtypeStruct((N, 128), jnp.float32)] * 3,
        scratch_types=[
            pltpu.VMEM((n_ch, CH), jnp.int32),
            pltpu.VMEM((CH, 128), jnp.float32),
            pltpu.VMEM((CH, 128), jnp.float32),
            pltpu.VMEM((CH, 128), jnp.float32),
            pltpu.SemaphoreType.DMA,
            pltpu.SemaphoreType.DMA,
            pltpu.SemaphoreType.DMA,
        ],
    )
    def gather(l0, l1, l2, idx_hbm, o0, o1, o2, idx_v, b0, b1, b2, s0, s1, s2):
        wid = lax.axis_index("s") * NC + lax.axis_index("c")
        base = wid * per_w
        for ch in range(n_ch):
            pltpu.sync_copy(idx_hbm.at[pl.ds(base + ch * CH, CH)], idx_v.at[ch])
        for ch in range(n_ch):
            c0 = pltpu.async_copy(l0.at[idx_v.at[ch]], b0, s0)
            c1 = pltpu.async_copy(l1.at[idx_v.at[ch]], b1, s1)
            c2 = pltpu.async_copy(l2.at[idx_v.at[ch]], b2, s2)
            c0.wait()
            c1.wait()
            c2.wait()
            pltpu.sync_copy(b0, o0.at[pl.ds(base + ch * CH, CH)])
            pltpu.sync_copy(b1, o1.at[pl.ds(base + ch * CH, CH)])
            pltpu.sync_copy(b2, o2.at[pl.ds(base + ch * CH, CH)])

    return gather


def _detile_body(et_ref, o0_ref, o1_ref, o2_ref):
    t = et_ref[...]
    o0_ref[...] = jnp.transpose(t[0:128, :])
    o1_ref[...] = jnp.transpose(t[128:256, :])
    o2_ref[...] = jnp.transpose(t[172:300, :])


def _detile(embT, row_block=2048):
    D, V = embT.shape
    grid = (pl.cdiv(V, row_block),)
    return pl.pallas_call(
        _detile_body,
        grid=grid,
        in_specs=[pl.BlockSpec((D, row_block), lambda i: (0, i))],
        out_specs=[pl.BlockSpec((row_block, 128), lambda i: (i, 0))] * 3,
        out_shape=[jax.ShapeDtypeStruct((V, 128), jnp.float32)] * 3,
    )(embT)


def _proj_body(e0_ref, e1_ref, e2_ref, w0_ref, w1_ref, w2_ref, b_ref, o_ref):
    bf = jnp.bfloat16
    acc = jnp.dot(e0_ref[...].astype(bf), w0_ref[...],
                  preferred_element_type=jnp.float32)
    acc += jnp.dot(e1_ref[...].astype(bf), w1_ref[...],
                   preferred_element_type=jnp.float32)
    acc += jnp.dot(e2_ref[...].astype(bf), w2_ref[...],
                   preferred_element_type=jnp.float32)
    o_ref[...] = (acc + b_ref[...]).astype(bf)


def _projection(planes, ws, bias, row_block=512):
    N = planes[0].shape[0]
    G4 = ws[0].shape[1]
    grid = (N // row_block,)
    return pl.pallas_call(
        _proj_body,
        grid=grid,
        in_specs=[pl.BlockSpec((row_block, 128), lambda i: (i, 0))] * 3
        + [pl.BlockSpec((128, G4), lambda i: (0, 0))] * 3
        + [pl.BlockSpec((1, G4), lambda i: (0, 0))],
        out_specs=pl.BlockSpec((row_block, G4), lambda i: (i, 0)),
        out_shape=jax.ShapeDtypeStruct((N, G4), jnp.bfloat16),
    )(*planes, *ws, bias)


def _rec_body(xp_ref, whh_ref, lw_ref, lb_ref, out_ref, h_ref, c_ref,
              *, H, B, U):
    t = pl.program_id(0)

    @pl.when(t == 0)
    def _init():
        h_ref[...] = jnp.zeros_like(h_ref)
        c_ref[...] = jnp.zeros_like(c_ref)

    h = h_ref[...]
    c = c_ref[...]
    for u in range(U):
        g = xp_ref[u * B : (u + 1) * B, :].astype(jnp.float32) + jnp.dot(
            h.astype(jnp.bfloat16), whh_ref[...],
            preferred_element_type=jnp.float32,
        )
        i = jax.nn.sigmoid(g[:, :H])
        f = jax.nn.sigmoid(g[:, H : 2 * H])
        gg = jnp.tanh(g[:, 2 * H : 3 * H])
        o = jax.nn.sigmoid(g[:, 3 * H :])
        c = f * c + i * gg
        h = o * jnp.tanh(c)
    c_ref[...] = c
    h_ref[...] = h

    @pl.when(t == pl.num_programs(0) - 1)
    def _head():
        out_ref[...] = (
            jnp.dot(h, lw_ref[...], preferred_element_type=jnp.float32)
            + lb_ref[...]
        )


def _recurrence(xp, whhT, lwT, lb, B, unroll=2):
    N, G4 = xp.shape
    S = N // B
    H = G4 // 4
    C = lwT.shape[1]
    return pl.pallas_call(
        functools.partial(_rec_body, H=H, B=B, U=unroll),
        grid=(S // unroll,),
        in_specs=[
            pl.BlockSpec((unroll * B, G4), lambda t: (t, 0)),
            pl.BlockSpec((H, G4), lambda t: (0, 0)),
            pl.BlockSpec((H, C), lambda t: (0, 0)),
            pl.BlockSpec((1, C), lambda t: (0, 0)),
        ],
        out_specs=pl.BlockSpec((B, C), lambda t: (0, 0)),
        out_shape=jax.ShapeDtypeStruct((B, C), jnp.float32),
        scratch_shapes=[
            pltpu.VMEM((B, H), jnp.float32),
            pltpu.VMEM((B, H), jnp.float32),
        ],
    )(xp, whhT, lwT, lb)


def kernel(x, emb, W_ih0, W_hh0, b_ih0, b_hh0,
           W_ih1, W_hh1, b_ih1, b_hh1, linW, linb):
    B, S = x.shape
    V, D = emb.shape
    G4, H = W_hh0.shape[0], W_hh0.shape[1]
    C = linW.shape[0]

    idx = jnp.transpose(x).reshape(-1)
    L0, L1, L2 = _detile(jnp.transpose(emb))
    O0, O1, O2 = _make_gather3(V, S * B)(L0, L1, L2, idx)

    wT = jnp.transpose(W_ih0)
    W0 = wT[0:128].astype(jnp.bfloat16)
    W1 = wT[128:256].astype(jnp.bfloat16)
    W2 = jnp.concatenate(
        [jnp.zeros((84, G4), wT.dtype), wT[256:300]], axis=0
    ).astype(jnp.bfloat16)
    bias = (b_ih0 + b_hh0).reshape(1, G4)
    xp = _projection((O0, O1, O2), (W0, W1, W2), bias)

    whhT = jnp.transpose(W_hh0).astype(jnp.bfloat16)
    lwT = jnp.transpose(linW)
    out = _recurrence(xp, whhT, lwT, linb.reshape(1, C), B)
    return out


# speedup vs baseline: 5.4625x; 1.0268x over previous
import functools

import jax
import jax.numpy as jnp
from jax import lax
from jax.experimental import pallas as pl
from jax.experimental.pallas import tpu as pltpu
from jax.experimental.pallas import tpu_sc as plsc


def _make_gather3(V, N):
    info = plsc.get_sparse_core_info()
    NC, NS = info.num_cores, info.num_subcores
    NW = NC * NS
    per_w = N // NW
    CH = min(128, per_w)
    n_ch = per_w // CH
    mesh = plsc.VectorSubcoreMesh(core_axis_name="c", subcore_axis_name="s")

    @functools.partial(
        pl.kernel,
        mesh=mesh,
        compiler_params=pltpu.CompilerParams(use_tc_tiling_on_sc=False),
        out_type=[jax.ShapeDtypeStruct((N, 128), jnp.float32)] * 3,
        scratch_types=[
            pltpu.VMEM((n_ch, CH), jnp.int32),
            pltpu.VMEM((CH, 128), jnp.float32),
            pltpu.VMEM((CH, 128), jnp.float32),
            pltpu.VMEM((CH, 128), jnp.float32),
            pltpu.SemaphoreType.DMA,
            pltpu.SemaphoreType.DMA,
            pltpu.SemaphoreType.DMA,
        ],
    )
    def gather(l0, l1, l2, idx_hbm, o0, o1, o2, idx_v, b0, b1, b2, s0, s1, s2):
        wid = lax.axis_index("s") * NC + lax.axis_index("c")
        base = wid * per_w
        for ch in range(n_ch):
            pltpu.sync_copy(idx_hbm.at[pl.ds(base + ch * CH, CH)], idx_v.at[ch])
        for ch in range(n_ch):
            c0 = pltpu.async_copy(l0.at[idx_v.at[ch]], b0, s0)
            c1 = pltpu.async_copy(l1.at[idx_v.at[ch]], b1, s1)
            c2 = pltpu.async_copy(l2.at[idx_v.at[ch]], b2, s2)
            c0.wait()
            c1.wait()
            c2.wait()
            pltpu.sync_copy(b0, o0.at[pl.ds(base + ch * CH, CH)])
            pltpu.sync_copy(b1, o1.at[pl.ds(base + ch * CH, CH)])
            pltpu.sync_copy(b2, o2.at[pl.ds(base + ch * CH, CH)])

    return gather


def _detile_body(et_ref, o0_ref, o1_ref, o2_ref):
    t = et_ref[...]
    o0_ref[...] = jnp.transpose(t[0:128, :])
    o1_ref[...] = jnp.transpose(t[128:256, :])
    o2_ref[...] = jnp.transpose(t[172:300, :])


def _detile(embT, row_block=2048):
    D, V = embT.shape
    grid = (pl.cdiv(V, row_block),)
    return pl.pallas_call(
        _detile_body,
        grid=grid,
        in_specs=[pl.BlockSpec((D, row_block), lambda i: (0, i))],
        out_specs=[pl.BlockSpec((row_block, 128), lambda i: (i, 0))] * 3,
        out_shape=[jax.ShapeDtypeStruct((V, 128), jnp.float32)] * 3,
    )(embT)


def _proj_body(e0_ref, e1_ref, e2_ref, w0_ref, w1_ref, w2_ref, b_ref, o_ref):
    bf = jnp.bfloat16
    acc = jnp.dot(e0_ref[...].astype(bf), w0_ref[...],
                  preferred_element_type=jnp.float32)
    acc += jnp.dot(e1_ref[...].astype(bf), w1_ref[...],
                   preferred_element_type=jnp.float32)
    acc += jnp.dot(e2_ref[...].astype(bf), w2_ref[...],
                   preferred_element_type=jnp.float32)
    o_ref[...] = (acc + b_ref[...]).astype(bf)


def _projection(planes, ws, bias, row_block=512):
    N = planes[0].shape[0]
    G4 = ws[0].shape[1]
    grid = (N // row_block,)
    return pl.pallas_call(
        _proj_body,
        grid=grid,
        in_specs=[pl.BlockSpec((row_block, 128), lambda i: (i, 0))] * 3
        + [pl.BlockSpec((128, G4), lambda i: (0, 0))] * 3
        + [pl.BlockSpec((1, G4), lambda i: (0, 0))],
        out_specs=pl.BlockSpec((row_block, G4), lambda i: (i, 0)),
        out_shape=jax.ShapeDtypeStruct((N, G4), jnp.bfloat16),
    )(*planes, *ws, bias)


def _rec_body(xp_ref, whh_ref, lw_ref, lb_ref, out_ref, h_ref, c_ref,
              *, H, B, U):
    t = pl.program_id(0)

    @pl.when(t == 0)
    def _init():
        h_ref[...] = jnp.zeros_like(h_ref)
        c_ref[...] = jnp.zeros_like(c_ref)

    h = h_ref[...]
    c = c_ref[...]
    for u in range(U):
        g = xp_ref[u * B : (u + 1) * B, :].astype(jnp.float32) + jnp.dot(
            h.astype(jnp.bfloat16), whh_ref[...],
            preferred_element_type=jnp.float32,
        )
        i = jax.nn.sigmoid(g[:, :H])
        f = jax.nn.sigmoid(g[:, H : 2 * H])
        gg = jnp.tanh(g[:, 2 * H : 3 * H])
        o = jax.nn.sigmoid(g[:, 3 * H :])
        c = f * c + i * gg
        h = o * jnp.tanh(c)
    c_ref[...] = c
    h_ref[...] = h

    @pl.when(t == pl.num_programs(0) - 1)
    def _head():
        out_ref[...] = (
            jnp.dot(h, lw_ref[...], preferred_element_type=jnp.float32)
            + lb_ref[...]
        )


def _recurrence(xp, whhT, lwT, lb, B, unroll=4):
    N, G4 = xp.shape
    S = N // B
    H = G4 // 4
    C = lwT.shape[1]
    return pl.pallas_call(
        functools.partial(_rec_body, H=H, B=B, U=unroll),
        grid=(S // unroll,),
        in_specs=[
            pl.BlockSpec((unroll * B, G4), lambda t: (t, 0)),
            pl.BlockSpec((H, G4), lambda t: (0, 0)),
            pl.BlockSpec((H, C), lambda t: (0, 0)),
            pl.BlockSpec((1, C), lambda t: (0, 0)),
        ],
        out_specs=pl.BlockSpec((B, C), lambda t: (0, 0)),
        out_shape=jax.ShapeDtypeStruct((B, C), jnp.float32),
        scratch_shapes=[
            pltpu.VMEM((B, H), jnp.float32),
            pltpu.VMEM((B, H), jnp.float32),
        ],
    )(xp, whhT, lwT, lb)


def kernel(x, emb, W_ih0, W_hh0, b_ih0, b_hh0,
           W_ih1, W_hh1, b_ih1, b_hh1, linW, linb):
    B, S = x.shape
    V, D = emb.shape
    G4, H = W_hh0.shape[0], W_hh0.shape[1]
    C = linW.shape[0]

    idx = jnp.transpose(x).reshape(-1)
    L0, L1, L2 = _detile(jnp.transpose(emb))
    O0, O1, O2 = _make_gather3(V, S * B)(L0, L1, L2, idx)

    wT = jnp.transpose(W_ih0)
    W0 = wT[0:128].astype(jnp.bfloat16)
    W1 = wT[128:256].astype(jnp.bfloat16)
    W2 = jnp.concatenate(
        [jnp.zeros((84, G4), wT.dtype), wT[256:300]], axis=0
    ).astype(jnp.bfloat16)
    bias = (b_ih0 + b_hh0).reshape(1, G4)
    xp = _projection((O0, O1, O2), (W0, W1, W2), bias)

    whhT = jnp.transpose(W_hh0).astype(jnp.bfloat16)
    lwT = jnp.transpose(linW)
    out = _recurrence(xp, whhT, lwT, linb.reshape(1, C), B)
    return out


# speedup vs baseline: 5.4865x; 1.0044x over previous
import functools

import jax
import jax.numpy as jnp
from jax import lax
from jax.experimental import pallas as pl
from jax.experimental.pallas import tpu as pltpu
from jax.experimental.pallas import tpu_sc as plsc


def _make_gather3(V, N):
    info = plsc.get_sparse_core_info()
    NC, NS = info.num_cores, info.num_subcores
    NW = NC * NS
    per_w = N // NW
    CH = min(128, per_w)
    n_ch = per_w // CH
    mesh = plsc.VectorSubcoreMesh(core_axis_name="c", subcore_axis_name="s")

    @functools.partial(
        pl.kernel,
        mesh=mesh,
        compiler_params=pltpu.CompilerParams(use_tc_tiling_on_sc=False),
        out_type=[jax.ShapeDtypeStruct((N, 128), jnp.float32)] * 3,
        scratch_types=[
            pltpu.VMEM((n_ch, CH), jnp.int32),
            pltpu.VMEM((per_w, 128), jnp.float32),
            pltpu.VMEM((per_w, 128), jnp.float32),
            pltpu.VMEM((per_w, 128), jnp.float32),
            pltpu.SemaphoreType.DMA,
            pltpu.SemaphoreType.DMA,
            pltpu.SemaphoreType.DMA,
        ],
    )
    def gather(l0, l1, l2, idx_hbm, o0, o1, o2, idx_v, b0, b1, b2, s0, s1, s2):
        wid = lax.axis_index("s") * NC + lax.axis_index("c")
        base = wid * per_w
        for ch in range(n_ch):
            pltpu.sync_copy(idx_hbm.at[pl.ds(base + ch * CH, CH)], idx_v.at[ch])
        waits = []
        for ch in range(n_ch):
            sl = pl.ds(ch * CH, CH)
            waits.append(pltpu.async_copy(l0.at[idx_v.at[ch]], b0.at[sl], s0))
            waits.append(pltpu.async_copy(l1.at[idx_v.at[ch]], b1.at[sl], s1))
            waits.append(pltpu.async_copy(l2.at[idx_v.at[ch]], b2.at[sl], s2))
        for c in waits:
            c.wait()
        pltpu.sync_copy(b0, o0.at[pl.ds(base, per_w)])
        pltpu.sync_copy(b1, o1.at[pl.ds(base, per_w)])
        pltpu.sync_copy(b2, o2.at[pl.ds(base, per_w)])

    return gather


def _detile_body(et_ref, o0_ref, o1_ref, o2_ref):
    t = et_ref[...]
    o0_ref[...] = jnp.transpose(t[0:128, :])
    o1_ref[...] = jnp.transpose(t[128:256, :])
    o2_ref[...] = jnp.transpose(t[172:300, :])


def _detile(embT, row_block=2048):
    D, V = embT.shape
    grid = (pl.cdiv(V, row_block),)
    return pl.pallas_call(
        _detile_body,
        grid=grid,
        in_specs=[pl.BlockSpec((D, row_block), lambda i: (0, i))],
        out_specs=[pl.BlockSpec((row_block, 128), lambda i: (i, 0))] * 3,
        out_shape=[jax.ShapeDtypeStruct((V, 128), jnp.float32)] * 3,
    )(embT)


def _proj_body(e0_ref, e1_ref, e2_ref, w0_ref, w1_ref, w2_ref, b_ref, o_ref):
    bf = jnp.bfloat16
    acc = jnp.dot(e0_ref[...].astype(bf), w0_ref[...],
                  preferred_element_type=jnp.float32)
    acc += jnp.dot(e1_ref[...].astype(bf), w1_ref[...],
                   preferred_element_type=jnp.float32)
    acc += jnp.dot(e2_ref[...].astype(bf), w2_ref[...],
                   preferred_element_type=jnp.float32)
    o_ref[...] = (acc + b_ref[...]).astype(bf)


def _projection(planes, ws, bias, row_block=512):
    N = planes[0].shape[0]
    G4 = ws[0].shape[1]
    grid = (N // row_block,)
    return pl.pallas_call(
        _proj_body,
        grid=grid,
        in_specs=[pl.BlockSpec((row_block, 128), lambda i: (i, 0))] * 3
        + [pl.BlockSpec((128, G4), lambda i: (0, 0))] * 3
        + [pl.BlockSpec((1, G4), lambda i: (0, 0))],
        out_specs=pl.BlockSpec((row_block, G4), lambda i: (i, 0)),
        out_shape=jax.ShapeDtypeStruct((N, G4), jnp.bfloat16),
    )(*planes, *ws, bias)


def _rec_body(xp_ref, whh_ref, lw_ref, lb_ref, out_ref, h_ref, c_ref,
              *, H, B, U):
    t = pl.program_id(0)

    @pl.when(t == 0)
    def _init():
        h_ref[...] = jnp.zeros_like(h_ref)
        c_ref[...] = jnp.zeros_like(c_ref)

    h = h_ref[...]
    c = c_ref[...]
    for u in range(U):
        g = xp_ref[u * B : (u + 1) * B, :].astype(jnp.float32) + jnp.dot(
            h.astype(jnp.bfloat16), whh_ref[...],
            preferred_element_type=jnp.float32,
        )
        i = jax.nn.sigmoid(g[:, :H])
        f = jax.nn.sigmoid(g[:, H : 2 * H])
        gg = jnp.tanh(g[:, 2 * H : 3 * H])
        o = jax.nn.sigmoid(g[:, 3 * H :])
        c = f * c + i * gg
        h = o * jnp.tanh(c)
    c_ref[...] = c
    h_ref[...] = h

    @pl.when(t == pl.num_programs(0) - 1)
    def _head():
        out_ref[...] = (
            jnp.dot(h, lw_ref[...], preferred_element_type=jnp.float32)
            + lb_ref[...]
        )


def _recurrence(xp, whhT, lwT, lb, B, unroll=4):
    N, G4 = xp.shape
    S = N // B
    H = G4 // 4
    C = lwT.shape[1]
    return pl.pallas_call(
        functools.partial(_rec_body, H=H, B=B, U=unroll),
        grid=(S // unroll,),
        in_specs=[
            pl.BlockSpec((unroll * B, G4), lambda t: (t, 0)),
            pl.BlockSpec((H, G4), lambda t: (0, 0)),
            pl.BlockSpec((H, C), lambda t: (0, 0)),
            pl.BlockSpec((1, C), lambda t: (0, 0)),
        ],
        out_specs=pl.BlockSpec((B, C), lambda t: (0, 0)),
        out_shape=jax.ShapeDtypeStruct((B, C), jnp.float32),
        scratch_shapes=[
            pltpu.VMEM((B, H), jnp.float32),
            pltpu.VMEM((B, H), jnp.float32),
        ],
    )(xp, whhT, lwT, lb)


def kernel(x, emb, W_ih0, W_hh0, b_ih0, b_hh0,
           W_ih1, W_hh1, b_ih1, b_hh1, linW, linb):
    B, S = x.shape
    V, D = emb.shape
    G4, H = W_hh0.shape[0], W_hh0.shape[1]
    C = linW.shape[0]

    idx = jnp.transpose(x).reshape(-1)
    L0, L1, L2 = _detile(jnp.transpose(emb))
    O0, O1, O2 = _make_gather3(V, S * B)(L0, L1, L2, idx)

    wT = jnp.transpose(W_ih0)
    W0 = wT[0:128].astype(jnp.bfloat16)
    W1 = wT[128:256].astype(jnp.bfloat16)
    W2 = jnp.concatenate(
        [jnp.zeros((84, G4), wT.dtype), wT[256:300]], axis=0
    ).astype(jnp.bfloat16)
    bias = (b_ih0 + b_hh0).reshape(1, G4)
    xp = _projection((O0, O1, O2), (W0, W1, W2), bias)

    whhT = jnp.transpose(W_hh0).astype(jnp.bfloat16)
    lwT = jnp.transpose(linW)
    out = _recurrence(xp, whhT, lwT, linb.reshape(1, C), B)
    return out


# speedup vs baseline: 5.7033x; 1.0395x over previous
import functools

import jax
import jax.numpy as jnp
from jax import lax
from jax.experimental import pallas as pl
from jax.experimental.pallas import tpu as pltpu
from jax.experimental.pallas import tpu_sc as plsc


def _make_gather3(V, N):
    info = plsc.get_sparse_core_info()
    NC, NS = info.num_cores, info.num_subcores
    NW = NC * NS
    per_w = N // NW
    CH = min(128, per_w)
    n_ch = per_w // CH
    mesh = plsc.VectorSubcoreMesh(core_axis_name="c", subcore_axis_name="s")

    @functools.partial(
        pl.kernel,
        mesh=mesh,
        compiler_params=pltpu.CompilerParams(use_tc_tiling_on_sc=False),
        out_type=[jax.ShapeDtypeStruct((N, 128), jnp.float32)] * 3,
        scratch_types=[
            pltpu.VMEM((n_ch, CH), jnp.int32),
            pltpu.VMEM((per_w, 128), jnp.float32),
            pltpu.VMEM((per_w, 128), jnp.float32),
            pltpu.VMEM((per_w, 128), jnp.float32),
            pltpu.SemaphoreType.DMA,
            pltpu.SemaphoreType.DMA,
            pltpu.SemaphoreType.DMA,
        ],
    )
    def gather(l0, l1, l2, idx_hbm, o0, o1, o2, idx_v, b0, b1, b2, s0, s1, s2):
        wid = lax.axis_index("s") * NC + lax.axis_index("c")
        base = wid * per_w
        for ch in range(n_ch):
            pltpu.sync_copy(idx_hbm.at[pl.ds(base + ch * CH, CH)], idx_v.at[ch])
        waits = []
        for ch in range(n_ch):
            sl = pl.ds(ch * CH, CH)
            waits.append(pltpu.async_copy(l0.at[idx_v.at[ch]], b0.at[sl], s0))
            waits.append(pltpu.async_copy(l1.at[idx_v.at[ch]], b1.at[sl], s1))
            waits.append(pltpu.async_copy(l2.at[idx_v.at[ch]], b2.at[sl], s2))
        for c in waits:
            c.wait()
        pltpu.sync_copy(b0, o0.at[pl.ds(base, per_w)])
        pltpu.sync_copy(b1, o1.at[pl.ds(base, per_w)])
        pltpu.sync_copy(b2, o2.at[pl.ds(base, per_w)])

    return gather


def _detile_body(et_ref, o0_ref, o1_ref, o2_ref):
    t = et_ref[...]
    o0_ref[...] = jnp.transpose(t[0:128, :])
    o1_ref[...] = jnp.transpose(t[128:256, :])
    o2_ref[...] = jnp.transpose(t[172:300, :])


def _detile(embT, row_block=4096):
    D, V = embT.shape
    grid = (pl.cdiv(V, row_block),)
    return pl.pallas_call(
        _detile_body,
        grid=grid,
        in_specs=[pl.BlockSpec((D, row_block), lambda i: (0, i))],
        out_specs=[pl.BlockSpec((row_block, 128), lambda i: (i, 0))] * 3,
        out_shape=[jax.ShapeDtypeStruct((V, 128), jnp.float32)] * 3,
    )(embT)


def _proj_body(e0_ref, e1_ref, e2_ref, w0_ref, w1_ref, w2_ref, b_ref, o_ref):
    bf = jnp.bfloat16
    acc = jnp.dot(e0_ref[...].astype(bf), w0_ref[...],
                  preferred_element_type=jnp.float32)
    acc += jnp.dot(e1_ref[...].astype(bf), w1_ref[...],
                   preferred_element_type=jnp.float32)
    acc += jnp.dot(e2_ref[...].astype(bf), w2_ref[...],
                   preferred_element_type=jnp.float32)
    o_ref[...] = (acc + b_ref[...]).astype(bf)


def _projection(planes, ws, bias, row_block=1024):
    N = planes[0].shape[0]
    G4 = ws[0].shape[1]
    grid = (N // row_block,)
    return pl.pallas_call(
        _proj_body,
        grid=grid,
        in_specs=[pl.BlockSpec((row_block, 128), lambda i: (i, 0))] * 3
        + [pl.BlockSpec((128, G4), lambda i: (0, 0))] * 3
        + [pl.BlockSpec((1, G4), lambda i: (0, 0))],
        out_specs=pl.BlockSpec((row_block, G4), lambda i: (i, 0)),
        out_shape=jax.ShapeDtypeStruct((N, G4), jnp.bfloat16),
    )(*planes, *ws, bias)


def _rec_body(xp_ref, whh_ref, lw_ref, lb_ref, out_ref, h_ref, c_ref,
              *, H, B, U):
    t = pl.program_id(0)

    @pl.when(t == 0)
    def _init():
        h_ref[...] = jnp.zeros_like(h_ref)
        c_ref[...] = jnp.zeros_like(c_ref)

    h = h_ref[...]
    c = c_ref[...]
    for u in range(U):
        g = xp_ref[u * B : (u + 1) * B, :].astype(jnp.float32) + jnp.dot(
            h.astype(jnp.bfloat16), whh_ref[...],
            preferred_element_type=jnp.float32,
        )
        i = jax.nn.sigmoid(g[:, :H])
        f = jax.nn.sigmoid(g[:, H : 2 * H])
        gg = jnp.tanh(g[:, 2 * H : 3 * H])
        o = jax.nn.sigmoid(g[:, 3 * H :])
        c = f * c + i * gg
        h = o * jnp.tanh(c)
    c_ref[...] = c
    h_ref[...] = h

    @pl.when(t == pl.num_programs(0) - 1)
    def _head():
        out_ref[...] = (
            jnp.dot(h, lw_ref[...], preferred_element_type=jnp.float32)
            + lb_ref[...]
        )


def _recurrence(xp, whhT, lwT, lb, B, unroll=8):
    N, G4 = xp.shape
    S = N // B
    H = G4 // 4
    C = lwT.shape[1]
    return pl.pallas_call(
        functools.partial(_rec_body, H=H, B=B, U=unroll),
        grid=(S // unroll,),
        in_specs=[
            pl.BlockSpec((unroll * B, G4), lambda t: (t, 0)),
            pl.BlockSpec((H, G4), lambda t: (0, 0)),
            pl.BlockSpec((H, C), lambda t: (0, 0)),
            pl.BlockSpec((1, C), lambda t: (0, 0)),
        ],
        out_specs=pl.BlockSpec((B, C), lambda t: (0, 0)),
        out_shape=jax.ShapeDtypeStruct((B, C), jnp.float32),
        scratch_shapes=[
            pltpu.VMEM((B, H), jnp.float32),
            pltpu.VMEM((B, H), jnp.float32),
        ],
    )(xp, whhT, lwT, lb)


def kernel(x, emb, W_ih0, W_hh0, b_ih0, b_hh0,
           W_ih1, W_hh1, b_ih1, b_hh1, linW, linb):
    B, S = x.shape
    V, D = emb.shape
    G4, H = W_hh0.shape[0], W_hh0.shape[1]
    C = linW.shape[0]

    idx = jnp.transpose(x).reshape(-1)
    L0, L1, L2 = _detile(jnp.transpose(emb))
    O0, O1, O2 = _make_gather3(V, S * B)(L0, L1, L2, idx)

    wT = jnp.transpose(W_ih0)
    W0 = wT[0:128].astype(jnp.bfloat16)
    W1 = wT[128:256].astype(jnp.bfloat16)
    W2 = jnp.concatenate(
        [jnp.zeros((84, G4), wT.dtype), wT[256:300]], axis=0
    ).astype(jnp.bfloat16)
    bias = (b_ih0 + b_hh0).reshape(1, G4)
    xp = _projection((O0, O1, O2), (W0, W1, W2), bias)

    whhT = jnp.transpose(W_hh0).astype(jnp.bfloat16)
    lwT = jnp.transpose(linW)
    out = _recurrence(xp, whhT, lwT, linb.reshape(1, C), B)
    return out


# speedup vs baseline: 5.7107x; 1.0013x over previous
import functools

import jax
import jax.numpy as jnp
from jax import lax
from jax.experimental import pallas as pl
from jax.experimental.pallas import tpu as pltpu
from jax.experimental.pallas import tpu_sc as plsc


def _make_gather3(V, N):
    info = plsc.get_sparse_core_info()
    NC, NS = info.num_cores, info.num_subcores
    NW = NC * NS
    per_w = N // NW
    CH = min(128, per_w)
    n_ch = per_w // CH
    mesh = plsc.VectorSubcoreMesh(core_axis_name="c", subcore_axis_name="s")

    @functools.partial(
        pl.kernel,
        mesh=mesh,
        compiler_params=pltpu.CompilerParams(use_tc_tiling_on_sc=False),
        out_type=[jax.ShapeDtypeStruct((N, 128), jnp.float32)] * 3,
        scratch_types=[
            pltpu.VMEM((n_ch, CH), jnp.int32),
            pltpu.VMEM((per_w, 128), jnp.float32),
            pltpu.VMEM((per_w, 128), jnp.float32),
            pltpu.VMEM((per_w, 128), jnp.float32),
            pltpu.SemaphoreType.DMA,
            pltpu.SemaphoreType.DMA,
            pltpu.SemaphoreType.DMA,
        ],
    )
    def gather(l0, l1, l2, idx_hbm, o0, o1, o2, idx_v, b0, b1, b2, s0, s1, s2):
        wid = lax.axis_index("s") * NC + lax.axis_index("c")
        base = wid * per_w
        for ch in range(n_ch):
            pltpu.sync_copy(idx_hbm.at[pl.ds(base + ch * CH, CH)], idx_v.at[ch])
        waits = []
        for ch in range(n_ch):
            sl = pl.ds(ch * CH, CH)
            waits.append(pltpu.async_copy(l0.at[idx_v.at[ch]], b0.at[sl], s0))
            waits.append(pltpu.async_copy(l1.at[idx_v.at[ch]], b1.at[sl], s1))
            waits.append(pltpu.async_copy(l2.at[idx_v.at[ch]], b2.at[sl], s2))
        for c in waits:
            c.wait()
        pltpu.sync_copy(b0, o0.at[pl.ds(base, per_w)])
        pltpu.sync_copy(b1, o1.at[pl.ds(base, per_w)])
        pltpu.sync_copy(b2, o2.at[pl.ds(base, per_w)])

    return gather


def _detile_body(et_ref, o0_ref, o1_ref, o2_ref):
    t = et_ref[...]
    o0_ref[...] = jnp.transpose(t[0:128, :])
    o1_ref[...] = jnp.transpose(t[128:256, :])
    o2_ref[...] = jnp.transpose(t[172:300, :])


def _detile(embT, row_block=4096):
    D, V = embT.shape
    grid = (pl.cdiv(V, row_block),)
    return pl.pallas_call(
        _detile_body,
        grid=grid,
        in_specs=[pl.BlockSpec((D, row_block), lambda i: (0, i))],
        out_specs=[pl.BlockSpec((row_block, 128), lambda i: (i, 0))] * 3,
        out_shape=[jax.ShapeDtypeStruct((V, 128), jnp.float32)] * 3,
    )(embT)


def _proj_body(e0_ref, e1_ref, e2_ref, w0_ref, w1_ref, w2_ref, b_ref, o_ref):
    bf = jnp.bfloat16
    acc = jnp.dot(e0_ref[...].astype(bf), w0_ref[...],
                  preferred_element_type=jnp.float32)
    acc += jnp.dot(e1_ref[...].astype(bf), w1_ref[...],
                   preferred_element_type=jnp.float32)
    acc += jnp.dot(e2_ref[...].astype(bf), w2_ref[...],
                   preferred_element_type=jnp.float32)
    o_ref[...] = (acc + b_ref[...]).astype(bf)


def _projection(planes, ws, bias, row_block=2048):
    N = planes[0].shape[0]
    G4 = ws[0].shape[1]
    grid = (N // row_block,)
    return pl.pallas_call(
        _proj_body,
        grid=grid,
        in_specs=[pl.BlockSpec((row_block, 128), lambda i: (i, 0))] * 3
        + [pl.BlockSpec((128, G4), lambda i: (0, 0))] * 3
        + [pl.BlockSpec((1, G4), lambda i: (0, 0))],
        out_specs=pl.BlockSpec((row_block, G4), lambda i: (i, 0)),
        out_shape=jax.ShapeDtypeStruct((N, G4), jnp.bfloat16),
    )(*planes, *ws, bias)


def _rec_body(xp_ref, whh_ref, lw_ref, lb_ref, out_ref, h_ref, c_ref,
              *, H, B, U):
    t = pl.program_id(0)

    @pl.when(t == 0)
    def _init():
        h_ref[...] = jnp.zeros_like(h_ref)
        c_ref[...] = jnp.zeros_like(c_ref)

    h = h_ref[...]
    c = c_ref[...]
    for u in range(U):
        g = xp_ref[u * B : (u + 1) * B, :].astype(jnp.float32) + jnp.dot(
            h.astype(jnp.bfloat16), whh_ref[...],
            preferred_element_type=jnp.float32,
        )
        i = jax.nn.sigmoid(g[:, :H])
        f = jax.nn.sigmoid(g[:, H : 2 * H])
        gg = jnp.tanh(g[:, 2 * H : 3 * H])
        o = jax.nn.sigmoid(g[:, 3 * H :])
        c = f * c + i * gg
        h = o * jnp.tanh(c)
    c_ref[...] = c
    h_ref[...] = h

    @pl.when(t == pl.num_programs(0) - 1)
    def _head():
        out_ref[...] = (
            jnp.dot(h, lw_ref[...], preferred_element_type=jnp.float32)
            + lb_ref[...]
        )


def _recurrence(xp, whhT, lwT, lb, B, unroll=16):
    N, G4 = xp.shape
    S = N // B
    H = G4 // 4
    C = lwT.shape[1]
    return pl.pallas_call(
        functools.partial(_rec_body, H=H, B=B, U=unroll),
        grid=(S // unroll,),
        in_specs=[
            pl.BlockSpec((unroll * B, G4), lambda t: (t, 0)),
            pl.BlockSpec((H, G4), lambda t: (0, 0)),
            pl.BlockSpec((H, C), lambda t: (0, 0)),
            pl.BlockSpec((1, C), lambda t: (0, 0)),
        ],
        out_specs=pl.BlockSpec((B, C), lambda t: (0, 0)),
        out_shape=jax.ShapeDtypeStruct((B, C), jnp.float32),
        scratch_shapes=[
            pltpu.VMEM((B, H), jnp.float32),
            pltpu.VMEM((B, H), jnp.float32),
        ],
    )(xp, whhT, lwT, lb)


def kernel(x, emb, W_ih0, W_hh0, b_ih0, b_hh0,
           W_ih1, W_hh1, b_ih1, b_hh1, linW, linb):
    B, S = x.shape
    V, D = emb.shape
    G4, H = W_hh0.shape[0], W_hh0.shape[1]
    C = linW.shape[0]

    idx = jnp.transpose(x).reshape(-1)
    L0, L1, L2 = _detile(jnp.transpose(emb))
    O0, O1, O2 = _make_gather3(V, S * B)(L0, L1, L2, idx)

    wT = jnp.transpose(W_ih0)
    W0 = wT[0:128].astype(jnp.bfloat16)
    W1 = wT[128:256].astype(jnp.bfloat16)
    W2 = jnp.concatenate(
        [jnp.zeros((84, G4), wT.dtype), wT[256:300]], axis=0
    ).astype(jnp.bfloat16)
    bias = (b_ih0 + b_hh0).reshape(1, G4)
    xp = _projection((O0, O1, O2), (W0, W1, W2), bias)

    whhT = jnp.transpose(W_hh0).astype(jnp.bfloat16)
    lwT = jnp.transpose(linW)
    out = _recurrence(xp, whhT, lwT, linb.reshape(1, C), B)
    return out
